# baseline (speedup 1.0000x reference)
import jax
import jax.numpy as jnp
from jax.experimental import pallas as pl
from jax.experimental.pallas import tpu as pltpu

N_BUF = 8
WIN = 32
TILE_E = 4096
N_CORES = 2


def _p12_kernel(h_ref, wc_ref, bc_ref, p1_ref, p2_ref):
    z = jnp.dot(h_ref[...], wc_ref[...],
                preferred_element_type=jnp.float32) + bc_ref[...]
    hblk = h_ref.shape[0]
    hp = p1_ref.shape[2]
    p1_ref[...] = z[:, :hp].reshape(hblk, 1, hp)
    p2_ref[...] = z[:, hp:].reshape(hblk, 1, hp)


def _scatter_kernel(src_ref, dst_ref, p1_ref, p2_ref, out_ref, *bufs):
    t = pl.program_id(1)

    @pl.when(t == 0)
    def _():
        for b in bufs:
            b[...] = jnp.zeros_like(b)

    def body(w, carry):
        base = w * (N_BUF * WIN)
        for sub in range(WIN):
            o = base + sub * N_BUF
            upd = []
            for k in range(N_BUF):
                s = src_ref[0, 0, o + k]
                d = dst_ref[0, 0, o + k]
                v = bufs[k][d] + (p1_ref[s] + p2_ref[d])
                upd.append((d, v))
            for k in range(N_BUF):
                d, v = upd[k]
                bufs[k][d] = v
        return carry

    jax.lax.fori_loop(0, TILE_E // (N_BUF * WIN), body, 0)

    @pl.when(t == pl.num_programs(1) - 1)
    def _():
        n = out_ref.shape[0]
        acc = bufs[0][:n]
        for b in bufs[1:]:
            acc = acc + b[:n]
        out_ref[...] = acc


def _node_kernel(h_ref, mp_ref, wn1_ref, wn2_ref, bn_ref, hnew_ref, m_ref):
    m = mp_ref[0] + mp_ref[1]
    z = (jnp.dot(h_ref[...], wn1_ref[...], preferred_element_type=jnp.float32)
         + jnp.dot(m, wn2_ref[...], preferred_element_type=jnp.float32)
         + bn_ref[...])
    hnew_ref[...] = jnp.maximum(z, 0.0)
    m_ref[...] = m


def kernel(h, src_idx, dst_idx, w_msg, b_msg, w_node, b_node):
    N, H = h.shape
    E = src_idx.shape[0]
    f32 = jnp.float32
    h = h.astype(f32)

    wcat = jnp.concatenate([w_msg[:H], w_msg[H:]], axis=1).astype(f32)
    bcat = jnp.concatenate([jnp.zeros((H,), f32),
                            b_msg.astype(f32)]).reshape(1, 2 * H)
    nblk = N // N_CORES
    p1_3d, p2_3d = pl.pallas_call(
        _p12_kernel,
        out_shape=(jax.ShapeDtypeStruct((N, 1, H), f32),
                   jax.ShapeDtypeStruct((N, 1, H), f32)),
        grid=(N_CORES,),
        in_specs=[
            pl.BlockSpec((nblk, H), lambda c: (c, 0)),
            pl.BlockSpec((H, 2 * H), lambda c: (0, 0)),
            pl.BlockSpec((1, 2 * H), lambda c: (0, 0)),
        ],
        out_specs=(pl.BlockSpec((nblk, 1, H), lambda c: (c, 0, 0)),
                   pl.BlockSpec((nblk, 1, H), lambda c: (c, 0, 0))),
        compiler_params=pltpu.CompilerParams(
            dimension_semantics=("parallel",)),
    )(h, wcat, bcat)

    chunk = N_CORES * TILE_E
    E_pad = (E + chunk - 1) // chunk * chunk
    pad_e = E_pad - E
    Nt = N + 8
    zrow = jnp.zeros((8, 1, H), f32)
    p1_t = jnp.concatenate([p1_3d, zrow], axis=0)
    p2_t = jnp.concatenate([p2_3d, zrow], axis=0)
    n_steps = E_pad // N_CORES // TILE_E
    src2 = jnp.pad(src_idx.astype(jnp.int32), (0, pad_e),
                   constant_values=N).reshape(N_CORES * n_steps, 1, TILE_E)
    dst2 = jnp.pad(dst_idx.astype(jnp.int32), (0, pad_e),
                   constant_values=N).reshape(N_CORES * n_steps, 1, TILE_E)

    m_part3 = pl.pallas_call(
        _scatter_kernel,
        out_shape=jax.ShapeDtypeStruct((N_CORES * N, 1, H), f32),
        grid=(N_CORES, n_steps),
        in_specs=[
            pl.BlockSpec((1, 1, TILE_E), lambda c, t: (c * n_steps + t, 0, 0),
                         memory_space=pltpu.SMEM),
            pl.BlockSpec((1, 1, TILE_E), lambda c, t: (c * n_steps + t, 0, 0),
                         memory_space=pltpu.SMEM),
            pl.BlockSpec((Nt, 1, H), lambda c, t: (0, 0, 0)),
            pl.BlockSpec((Nt, 1, H), lambda c, t: (0, 0, 0)),
        ],
        out_specs=pl.BlockSpec((N, 1, H), lambda c, t: (c, 0, 0)),
        scratch_shapes=[pltpu.VMEM((Nt, 1, H), f32) for _ in range(N_BUF)],
        compiler_params=pltpu.CompilerParams(
            dimension_semantics=("parallel", "arbitrary"),
            vmem_limit_bytes=60 * 2**20,
            disable_bounds_checks=True),
    )(src2, dst2, p1_t, p2_t)
    m_part = m_part3.reshape(N_CORES, N, H)

    wn1 = w_node[:H].astype(f32)
    wn2 = w_node[H:].astype(f32)
    bn = b_node.astype(f32).reshape(1, H)
    h_new, m = pl.pallas_call(
        _node_kernel,
        out_shape=(jax.ShapeDtypeStruct((N, H), f32),
                   jax.ShapeDtypeStruct((N, H), f32)),
        grid=(N_CORES,),
        in_specs=[
            pl.BlockSpec((nblk, H), lambda c: (c, 0)),
            pl.BlockSpec((N_CORES, nblk, H), lambda c: (0, c, 0)),
            pl.BlockSpec((H, H), lambda c: (0, 0)),
            pl.BlockSpec((H, H), lambda c: (0, 0)),
            pl.BlockSpec((1, H), lambda c: (0, 0)),
        ],
        out_specs=(pl.BlockSpec((nblk, H), lambda c: (c, 0)),
                   pl.BlockSpec((nblk, H), lambda c: (c, 0))),
        compiler_params=pltpu.CompilerParams(
            dimension_semantics=("parallel",)),
    )(h, m_part, wn1, wn2, bn)

    return h_new, m


# speedup vs baseline: 1.0095x; 1.0095x over previous
import jax
import jax.numpy as jnp
from jax.experimental import pallas as pl
from jax.experimental.pallas import tpu as pltpu

N_BUF = 8
WIN = 16
TILE_E = 8192
N_CORES = 2


def _p12_kernel(h_ref, wc_ref, bc_ref, p1_ref, p2_ref):
    z = jnp.dot(h_ref[...], wc_ref[...],
                preferred_element_type=jnp.float32) + bc_ref[...]
    hblk = h_ref.shape[0]
    hp = p1_ref.shape[2]
    p1_ref[...] = z[:, :hp].reshape(hblk, 1, hp)
    p2_ref[...] = z[:, hp:].reshape(hblk, 1, hp)


def _scatter_kernel(src_ref, dst_ref, p1_ref, p2_ref, out_ref, *bufs):
    t = pl.program_id(1)

    @pl.when(t == 0)
    def _():
        for b in bufs:
            b[...] = jnp.zeros_like(b)

    def body(w, carry):
        base = w * (N_BUF * WIN)
        for sub in range(WIN):
            o = base + sub * N_BUF
            upd = []
            for k in range(N_BUF):
                s = src_ref[0, 0, o + k]
                d = dst_ref[0, 0, o + k]
                v = bufs[k][d] + (p1_ref[s] + p2_ref[d])
                upd.append((d, v))
            for k in range(N_BUF):
                d, v = upd[k]
                bufs[k][d] = v
        return carry

    jax.lax.fori_loop(0, TILE_E // (N_BUF * WIN), body, 0)

    @pl.when(t == pl.num_programs(1) - 1)
    def _():
        n = out_ref.shape[0]
        acc = bufs[0][:n]
        for b in bufs[1:]:
            acc = acc + b[:n]
        out_ref[...] = acc


def _node_kernel(h_ref, mp_ref, wn1_ref, wn2_ref, bn_ref, hnew_ref, m_ref):
    m = mp_ref[0] + mp_ref[1]
    z = (jnp.dot(h_ref[...], wn1_ref[...], preferred_element_type=jnp.float32)
         + jnp.dot(m, wn2_ref[...], preferred_element_type=jnp.float32)
         + bn_ref[...])
    hnew_ref[...] = jnp.maximum(z, 0.0)
    m_ref[...] = m


def kernel(h, src_idx, dst_idx, w_msg, b_msg, w_node, b_node):
    N, H = h.shape
    E = src_idx.shape[0]
    f32 = jnp.float32
    h = h.astype(f32)

    wcat = jnp.concatenate([w_msg[:H], w_msg[H:]], axis=1).astype(f32)
    bcat = jnp.concatenate([jnp.zeros((H,), f32),
                            b_msg.astype(f32)]).reshape(1, 2 * H)
    nblk = N // N_CORES
    p1_3d, p2_3d = pl.pallas_call(
        _p12_kernel,
        out_shape=(jax.ShapeDtypeStruct((N, 1, H), f32),
                   jax.ShapeDtypeStruct((N, 1, H), f32)),
        grid=(N_CORES,),
        in_specs=[
            pl.BlockSpec((nblk, H), lambda c: (c, 0)),
            pl.BlockSpec((H, 2 * H), lambda c: (0, 0)),
            pl.BlockSpec((1, 2 * H), lambda c: (0, 0)),
        ],
        out_specs=(pl.BlockSpec((nblk, 1, H), lambda c: (c, 0, 0)),
                   pl.BlockSpec((nblk, 1, H), lambda c: (c, 0, 0))),
        compiler_params=pltpu.CompilerParams(
            dimension_semantics=("parallel",)),
    )(h, wcat, bcat)

    chunk = N_CORES * TILE_E
    E_pad = (E + chunk - 1) // chunk * chunk
    pad_e = E_pad - E
    Nt = N + 8
    zrow = jnp.zeros((8, 1, H), f32)
    p1_t = jnp.concatenate([p1_3d, zrow], axis=0)
    p2_t = jnp.concatenate([p2_3d, zrow], axis=0)
    n_steps = E_pad // N_CORES // TILE_E
    src2 = jnp.pad(src_idx.astype(jnp.int32), (0, pad_e),
                   constant_values=N).reshape(N_CORES * n_steps, 1, TILE_E)
    dst2 = jnp.pad(dst_idx.astype(jnp.int32), (0, pad_e),
                   constant_values=N).reshape(N_CORES * n_steps, 1, TILE_E)

    m_part3 = pl.pallas_call(
        _scatter_kernel,
        out_shape=jax.ShapeDtypeStruct((N_CORES * N, 1, H), f32),
        grid=(N_CORES, n_steps),
        in_specs=[
            pl.BlockSpec((1, 1, TILE_E), lambda c, t: (c * n_steps + t, 0, 0),
                         memory_space=pltpu.SMEM),
            pl.BlockSpec((1, 1, TILE_E), lambda c, t: (c * n_steps + t, 0, 0),
                         memory_space=pltpu.SMEM),
            pl.BlockSpec((Nt, 1, H), lambda c, t: (0, 0, 0)),
            pl.BlockSpec((Nt, 1, H), lambda c, t: (0, 0, 0)),
        ],
        out_specs=pl.BlockSpec((N, 1, H), lambda c, t: (c, 0, 0)),
        scratch_shapes=[pltpu.VMEM((Nt, 1, H), f32) for _ in range(N_BUF)],
        compiler_params=pltpu.CompilerParams(
            dimension_semantics=("parallel", "arbitrary"),
            vmem_limit_bytes=60 * 2**20,
            disable_bounds_checks=True),
    )(src2, dst2, p1_t, p2_t)
    m_part = m_part3.reshape(N_CORES, N, H)

    wn1 = w_node[:H].astype(f32)
    wn2 = w_node[H:].astype(f32)
    bn = b_node.astype(f32).reshape(1, H)
    h_new, m = pl.pallas_call(
        _node_kernel,
        out_shape=(jax.ShapeDtypeStruct((N, H), f32),
                   jax.ShapeDtypeStruct((N, H), f32)),
        grid=(N_CORES,),
        in_specs=[
            pl.BlockSpec((nblk, H), lambda c: (c, 0)),
            pl.BlockSpec((N_CORES, nblk, H), lambda c: (0, c, 0)),
            pl.BlockSpec((H, H), lambda c: (0, 0)),
            pl.BlockSpec((H, H), lambda c: (0, 0)),
            pl.BlockSpec((1, H), lambda c: (0, 0)),
        ],
        out_specs=(pl.BlockSpec((nblk, H), lambda c: (c, 0)),
                   pl.BlockSpec((nblk, H), lambda c: (c, 0))),
        compiler_params=pltpu.CompilerParams(
            dimension_semantics=("parallel",)),
    )(h, m_part, wn1, wn2, bn)

    return h_new, m
